# baseline (speedup 1.0000x reference)
import functools

import jax
import jax.numpy as jnp
from jax import lax
from jax.experimental import pallas as pl
from jax.experimental.pallas import tpu as pltpu
from jax.experimental.pallas import tpu_sc as plsc

NC = 2
NS = 16
NW = NC * NS
K = 8
BR = 8


@functools.cache
def _make_gather_kernel(S, V, D, BCH):
    NB = 16
    BB = BCH // NB
    SH = S // 2
    tpw = SH * BB
    mesh = plsc.VectorSubcoreMesh(
        core_axis_name="c", subcore_axis_name="s", num_cores=NC, num_subcores=NS
    )

    @functools.partial(
        pl.kernel,
        out_type=jax.ShapeDtypeStruct((S, BCH, D), jnp.float32),
        mesh=mesh,
        scratch_types=[
            pltpu.VMEM((tpw,), jnp.int32),
            [pltpu.VMEM((BB, D), jnp.float32)] * 2,
            [pltpu.SemaphoreType.DMA] * 2,
            [pltpu.SemaphoreType.DMA] * 2,
        ],
    )
    def gk(xu_ref, tok_ref, emb_ref, idx_v, bufs, gsems, osems):
        wid = lax.axis_index("s") * NC + lax.axis_index("c")
        half = wid // NB
        col = pl.multiple_of((wid % NB) * BB, BB)
        srow = half * SH

        pltpu.sync_copy(xu_ref.at[pl.ds(wid * tpw, tpw)], idx_v)

        def issue_gather(u, b):
            pltpu.async_copy(
                tok_ref.at[idx_v.at[pl.ds(u * BB, BB)]], bufs[b], gsems[b])

        def s_step(u, b, first=False):
            pltpu.make_async_copy(
                tok_ref.at[idx_v.at[pl.ds(u * BB, BB)]],
                bufs[b], gsems[b]).wait()
            pltpu.async_copy(
                bufs[b], emb_ref.at[srow + u, pl.ds(col, BB)], osems[b])

            if not first:
                pltpu.make_async_copy(
                    bufs[1 - b], emb_ref.at[0, pl.ds(0, BB)],
                    osems[1 - b]).wait()

            @pl.when(u + 1 < SH)
            def _():
                issue_gather(u + 1, 1 - b)

        issue_gather(0, 0)
        s_step(0, 0, first=True)

        def outer(o, _):
            for j in range(2):
                u = 1 + o * 2 + j
                s_step(u, 1 - j)
            return 0

        lax.fori_loop(0, (SH - 1) // 2, outer, 0)
        b_last = (SH - 1) % 2
        pltpu.make_async_copy(
            bufs[b_last], emb_ref.at[0, pl.ds(0, BB)], osems[b_last]).wait()

    return gk


def _ln_body(emb_ref, pos_ref, g_ref, b_ref, *rest):
    out_ref = rest[-1]
    S = emb_ref.shape[0]
    e = emb_ref[...] + pos_ref[0:S, :][:, None, :]
    mean = jnp.mean(e, axis=2, keepdims=True)
    c = e - mean
    var = jnp.mean(c * c, axis=2, keepdims=True)
    out_ref[...] = (c * lax.rsqrt(var + 1e-5) * g_ref[0][None, None, :]
                    + b_ref[0][None, None, :])


def _make_ln_call(k_idx, S, B, D, BCH, SP, aliased):
    nblk = BCH // BR
    base = k_idx * nblk
    out_spec = pl.BlockSpec((S, BR, D), lambda g: (0, base + g, 0))
    in_specs = [
        pl.BlockSpec((S, BR, D), lambda g: (0, g, 0)),
        pl.BlockSpec((SP, D), lambda g: (0, 0)),
        pl.BlockSpec((1, D), lambda g: (0, 0)),
        pl.BlockSpec((1, D), lambda g: (0, 0)),
    ]
    kwargs = {}
    if aliased:
        in_specs.append(pl.BlockSpec(memory_space=pl.ANY))
        kwargs["input_output_aliases"] = {4: 0}
    return pl.pallas_call(
        _ln_body,
        grid=(nblk,),
        in_specs=in_specs,
        out_specs=out_spec,
        out_shape=jax.ShapeDtypeStruct((S, B, D), jnp.float32),
        **kwargs,
    )


def kernel(x, tok_table, pos_table, gamma, beta):
    B, S = x.shape
    V, D = tok_table.shape
    SP = pos_table.shape[0]
    BCH = B // K
    g2 = gamma.reshape(1, D)
    b2 = beta.reshape(1, D)
    gk = _make_gather_kernel(S, V, D, BCH)
    out = None
    for k in range(K):
        xu = (x[k * BCH:(k + 1) * BCH].reshape(16, BCH // 16, 2, S // 2)
              .transpose(2, 0, 3, 1).reshape(-1))
        emb = gk(xu, tok_table)
        ln = _make_ln_call(k, S, B, D, BCH, SP, aliased=k > 0)
        args = (emb, pos_table, g2, b2) + ((out,) if k > 0 else ())
        out = ln(*args)
    return out.transpose(1, 0, 2)


# speedup vs baseline: 1.0707x; 1.0707x over previous
import functools

import jax
import jax.numpy as jnp
from jax import lax
from jax.experimental import pallas as pl
from jax.experimental.pallas import tpu as pltpu
from jax.experimental.pallas import tpu_sc as plsc

NC = 2
NS = 16
NW = NC * NS
K = 8
BR = 8


@functools.cache
def _make_gather_kernel(S, V, D, BCH):
    NB = 16
    BB = BCH // NB
    SH = S // 2
    tpw = SH * BB
    mesh = plsc.VectorSubcoreMesh(
        core_axis_name="c", subcore_axis_name="s", num_cores=NC, num_subcores=NS
    )

    @functools.partial(
        pl.kernel,
        out_type=jax.ShapeDtypeStruct((S, BCH, D), jnp.float32),
        mesh=mesh,
        scratch_types=[
            pltpu.VMEM((tpw,), jnp.int32),
            [pltpu.VMEM((BB, D), jnp.float32)] * 2,
            [pltpu.SemaphoreType.DMA] * 2,
            [pltpu.SemaphoreType.DMA] * 2,
        ],
    )
    def gk(xu_ref, tok_ref, emb_ref, idx_v, bufs, gsems, osems):
        wid = lax.axis_index("s") * NC + lax.axis_index("c")
        half = wid // NB
        col = pl.multiple_of((wid % NB) * BB, BB)
        srow = half * SH

        pltpu.sync_copy(xu_ref.at[pl.ds(wid * tpw, tpw)], idx_v)

        def issue_gather(u, b):
            pltpu.async_copy(
                tok_ref.at[idx_v.at[pl.ds(u * BB, BB)]], bufs[b], gsems[b])

        def s_step(u, b, first=False):
            pltpu.make_async_copy(
                tok_ref.at[idx_v.at[pl.ds(u * BB, BB)]],
                bufs[b], gsems[b]).wait()
            pltpu.async_copy(
                bufs[b], emb_ref.at[srow + u, pl.ds(col, BB)], osems[b])

            if not first:
                pltpu.make_async_copy(
                    bufs[1 - b], emb_ref.at[0, pl.ds(0, BB)],
                    osems[1 - b]).wait()

            @pl.when(u + 1 < SH)
            def _():
                issue_gather(u + 1, 1 - b)

        issue_gather(0, 0)
        s_step(0, 0, first=True)

        def outer(o, _):
            for j in range(2):
                u = 1 + o * 2 + j
                s_step(u, 1 - j)
            return 0

        lax.fori_loop(0, (SH - 1) // 2, outer, 0)
        b_last = (SH - 1) % 2
        pltpu.make_async_copy(
            bufs[b_last], emb_ref.at[0, pl.ds(0, BB)], osems[b_last]).wait()

    return gk


def _ln_body(emb_ref, pos_ref, g_ref, b_ref, *rest):
    out_ref = rest[-1]
    e = emb_ref[...] + pos_ref[...]
    mean = jnp.mean(e, axis=2, keepdims=True)
    c = e - mean
    var = jnp.mean(c * c, axis=2, keepdims=True)
    out_ref[...] = (c * lax.rsqrt(var + 1e-5) * g_ref[0][None, None, :]
                    + b_ref[0][None, None, :])


def _make_ln_call(k_idx, S, B, D, BCH, SP, aliased):
    out_spec = pl.BlockSpec((1, BCH, D), lambda g: (g, k_idx, 0))
    in_specs = [
        pl.BlockSpec((1, BCH, D), lambda g: (g, 0, 0)),
        pl.BlockSpec((1, 1, D), lambda g: (g, 0, 0)),
        pl.BlockSpec((1, D), lambda g: (0, 0)),
        pl.BlockSpec((1, D), lambda g: (0, 0)),
    ]
    kwargs = {}
    if aliased:
        in_specs.append(pl.BlockSpec(memory_space=pl.ANY))
        kwargs["input_output_aliases"] = {4: 0}
    return pl.pallas_call(
        _ln_body,
        grid=(S,),
        in_specs=in_specs,
        out_specs=out_spec,
        out_shape=jax.ShapeDtypeStruct((S, B, D), jnp.float32),
        **kwargs,
    )


def kernel(x, tok_table, pos_table, gamma, beta):
    B, S = x.shape
    V, D = tok_table.shape
    SP = pos_table.shape[0]
    BCH = B // K
    g2 = gamma.reshape(1, D)
    b2 = beta.reshape(1, D)
    pos3 = pos_table[:S].reshape(S, 1, D)
    gk = _make_gather_kernel(S, V, D, BCH)
    out = None
    for k in range(K):
        xu = (x[k * BCH:(k + 1) * BCH].reshape(16, BCH // 16, 2, S // 2)
              .transpose(2, 0, 3, 1).reshape(-1))
        emb = gk(xu, tok_table)
        ln = _make_ln_call(k, S, B, D, BCH, SP, aliased=k > 0)
        args = (emb, pos3, g2, b2) + ((out,) if k > 0 else ())
        out = ln(*args)
    return out.transpose(1, 0, 2)


# speedup vs baseline: 1.0717x; 1.0009x over previous
import functools

import jax
import jax.numpy as jnp
from jax import lax
from jax.experimental import pallas as pl
from jax.experimental.pallas import tpu as pltpu
from jax.experimental.pallas import tpu_sc as plsc

NC = 2
NS = 16
NW = NC * NS
K = 8
BR = 8


@functools.cache
def _make_gather_kernel(S, V, D, BCH):
    NB = 16
    BB = BCH // NB
    SH = S // 2
    tpw = SH * BB
    mesh = plsc.VectorSubcoreMesh(
        core_axis_name="c", subcore_axis_name="s", num_cores=NC, num_subcores=NS
    )

    @functools.partial(
        pl.kernel,
        out_type=jax.ShapeDtypeStruct((S, BCH, D), jnp.float32),
        mesh=mesh,
        scratch_types=[
            pltpu.VMEM((tpw,), jnp.int32),
            [pltpu.VMEM((BB, D), jnp.float32)] * 4,
            [pltpu.SemaphoreType.DMA] * 4,
            [pltpu.SemaphoreType.DMA] * 4,
        ],
    )
    def gk(xu_ref, tok_ref, emb_ref, idx_v, bufs, gsems, osems):
        wid = lax.axis_index("s") * NC + lax.axis_index("c")
        half = wid // NB
        col = pl.multiple_of((wid % NB) * BB, BB)
        srow = half * SH

        pltpu.sync_copy(xu_ref.at[pl.ds(wid * tpw, tpw)], idx_v)

        def issue_gather(u, b):
            pltpu.async_copy(
                tok_ref.at[idx_v.at[pl.ds(u * BB, BB)]], bufs[b], gsems[b])

        def wait_write(b):
            pltpu.make_async_copy(
                bufs[b], emb_ref.at[0, pl.ds(0, BB)], osems[b]).wait()

        def s_step(u, b, w_guard):
            pltpu.make_async_copy(
                tok_ref.at[idx_v.at[pl.ds(u * BB, BB)]],
                bufs[b], gsems[b]).wait()
            pltpu.async_copy(
                bufs[b], emb_ref.at[srow + u, pl.ds(col, BB)], osems[b])
            if w_guard == "skip":
                pass
            elif w_guard == "cond":
                @pl.when(u >= 2)
                def _():
                    wait_write((b + 2) % 4)
            else:
                wait_write((b + 2) % 4)

            @pl.when(u + 2 < SH)
            def _():
                issue_gather(u + 2, (b + 2) % 4)

        issue_gather(0, 0)
        issue_gather(1, 1)
        s_step(0, 0, "skip")

        def outer(o, _):
            for j in range(4):
                u = 1 + o * 4 + j
                s_step(u, (1 + j) % 4, "cond" if j == 0 else "go")
            return 0

        lax.fori_loop(0, (SH - 1) // 4, outer, 0)
        for u_last in (SH - 2, SH - 1):
            wait_write(u_last % 4)

    return gk


def _ln_body(emb_ref, pos_ref, g_ref, b_ref, *rest):
    out_ref = rest[-1]
    e = emb_ref[...] + pos_ref[...]
    mean = jnp.mean(e, axis=2, keepdims=True)
    c = e - mean
    var = jnp.mean(c * c, axis=2, keepdims=True)
    out_ref[...] = (c * lax.rsqrt(var + 1e-5) * g_ref[0][None, None, :]
                    + b_ref[0][None, None, :])


def _make_ln_call(k_idx, S, B, D, BCH, SP, aliased):
    out_spec = pl.BlockSpec((1, BCH, D), lambda g: (g, k_idx, 0))
    in_specs = [
        pl.BlockSpec((1, BCH, D), lambda g: (g, 0, 0)),
        pl.BlockSpec((1, 1, D), lambda g: (g, 0, 0)),
        pl.BlockSpec((1, D), lambda g: (0, 0)),
        pl.BlockSpec((1, D), lambda g: (0, 0)),
    ]
    kwargs = {}
    if aliased:
        in_specs.append(pl.BlockSpec(memory_space=pl.ANY))
        kwargs["input_output_aliases"] = {4: 0}
    return pl.pallas_call(
        _ln_body,
        grid=(S,),
        in_specs=in_specs,
        out_specs=out_spec,
        out_shape=jax.ShapeDtypeStruct((S, B, D), jnp.float32),
        **kwargs,
    )


def kernel(x, tok_table, pos_table, gamma, beta):
    B, S = x.shape
    V, D = tok_table.shape
    SP = pos_table.shape[0]
    BCH = B // K
    g2 = gamma.reshape(1, D)
    b2 = beta.reshape(1, D)
    pos3 = pos_table[:S].reshape(S, 1, D)
    gk = _make_gather_kernel(S, V, D, BCH)
    out = None
    for k in range(K):
        xu = (x[k * BCH:(k + 1) * BCH].reshape(16, BCH // 16, 2, S // 2)
              .transpose(2, 0, 3, 1).reshape(-1))
        emb = gk(xu, tok_table)
        ln = _make_ln_call(k, S, B, D, BCH, SP, aliased=k > 0)
        args = (emb, pos3, g2, b2) + ((out,) if k > 0 else ())
        out = ln(*args)
    return out.transpose(1, 0, 2)


# speedup vs baseline: 1.1323x; 1.0565x over previous
import functools

import jax
import jax.numpy as jnp
from jax import lax
from jax.experimental import pallas as pl
from jax.experimental.pallas import tpu as pltpu
from jax.experimental.pallas import tpu_sc as plsc

NC = 2
NS = 16
NW = NC * NS
K = 4
BR = 8


@functools.cache
def _make_gather_kernel(S, V, D, BCH):
    NB = 16
    BB = BCH // NB
    SH = S // 2
    tpw = SH * BB
    mesh = plsc.VectorSubcoreMesh(
        core_axis_name="c", subcore_axis_name="s", num_cores=NC, num_subcores=NS
    )

    @functools.partial(
        pl.kernel,
        out_type=jax.ShapeDtypeStruct((S, BCH, D), jnp.float32),
        mesh=mesh,
        scratch_types=[
            pltpu.VMEM((tpw,), jnp.int32),
            [pltpu.VMEM((BB, D), jnp.float32)] * 2,
            [pltpu.SemaphoreType.DMA] * 2,
            [pltpu.SemaphoreType.DMA] * 2,
        ],
    )
    def gk(xu_ref, tok_ref, emb_ref, idx_v, bufs, gsems, osems):
        wid = lax.axis_index("s") * NC + lax.axis_index("c")
        half = wid // NB
        col = pl.multiple_of((wid % NB) * BB, BB)
        srow = half * SH

        pltpu.sync_copy(xu_ref.at[pl.ds(wid * tpw, tpw)], idx_v)

        def issue_gather(u, b):
            pltpu.async_copy(
                tok_ref.at[idx_v.at[pl.ds(u * BB, BB)]], bufs[b], gsems[b])

        def s_step(u, b, first=False):
            pltpu.make_async_copy(
                tok_ref.at[idx_v.at[pl.ds(u * BB, BB)]],
                bufs[b], gsems[b]).wait()
            pltpu.async_copy(
                bufs[b], emb_ref.at[srow + u, pl.ds(col, BB)], osems[b])

            if not first:
                pltpu.make_async_copy(
                    bufs[1 - b], emb_ref.at[0, pl.ds(0, BB)],
                    osems[1 - b]).wait()

            @pl.when(u + 1 < SH)
            def _():
                issue_gather(u + 1, 1 - b)

        issue_gather(0, 0)
        s_step(0, 0, first=True)

        def outer(o, _):
            for j in range(2):
                u = 1 + o * 2 + j
                s_step(u, 1 - j)
            return 0

        lax.fori_loop(0, (SH - 1) // 2, outer, 0)
        b_last = (SH - 1) % 2
        pltpu.make_async_copy(
            bufs[b_last], emb_ref.at[0, pl.ds(0, BB)], osems[b_last]).wait()

    return gk


def _ln_body(emb_ref, pos_ref, g_ref, b_ref, *rest):
    out_ref = rest[-1]
    e = emb_ref[...] + pos_ref[...]
    mean = jnp.mean(e, axis=2, keepdims=True)
    c = e - mean
    var = jnp.mean(c * c, axis=2, keepdims=True)
    out_ref[...] = (c * lax.rsqrt(var + 1e-5) * g_ref[0][None, None, :]
                    + b_ref[0][None, None, :])


def _make_ln_call(k_idx, S, B, D, BCH, SP, aliased):
    out_spec = pl.BlockSpec((1, BCH, D), lambda g: (g, k_idx, 0))
    in_specs = [
        pl.BlockSpec((1, BCH, D), lambda g: (g, 0, 0)),
        pl.BlockSpec((1, 1, D), lambda g: (g, 0, 0)),
        pl.BlockSpec((1, D), lambda g: (0, 0)),
        pl.BlockSpec((1, D), lambda g: (0, 0)),
    ]
    kwargs = {}
    if aliased:
        in_specs.append(pl.BlockSpec(memory_space=pl.ANY))
        kwargs["input_output_aliases"] = {4: 0}
    return pl.pallas_call(
        _ln_body,
        grid=(S,),
        in_specs=in_specs,
        out_specs=out_spec,
        out_shape=jax.ShapeDtypeStruct((S, B, D), jnp.float32),
        **kwargs,
    )


def kernel(x, tok_table, pos_table, gamma, beta):
    B, S = x.shape
    V, D = tok_table.shape
    SP = pos_table.shape[0]
    BCH = B // K
    g2 = gamma.reshape(1, D)
    b2 = beta.reshape(1, D)
    pos3 = pos_table[:S].reshape(S, 1, D)
    gk = _make_gather_kernel(S, V, D, BCH)
    out = None
    for k in range(K):
        xu = (x[k * BCH:(k + 1) * BCH].reshape(16, BCH // 16, 2, S // 2)
              .transpose(2, 0, 3, 1).reshape(-1))
        emb = gk(xu, tok_table)
        ln = _make_ln_call(k, S, B, D, BCH, SP, aliased=k > 0)
        args = (emb, pos3, g2, b2) + ((out,) if k > 0 else ())
        out = ln(*args)
    return out.transpose(1, 0, 2)
